# baseline (speedup 1.0000x reference)
import functools

import jax
import jax.numpy as jnp
from jax import lax
from jax.experimental import pallas as pl
from jax.experimental.pallas import tpu as pltpu
from jax.experimental.pallas import tpu_sc as plsc

D = 768
H = 3072
E = 64
BLK = 64


def _gelu(v):
    return 0.5 * v * (1.0 + lax.erf(v * 0.7071067811865476))


def _gate_body(x_ref, gw_ref, gb_ref, bias_ref, eid_ref, w_ref):
    logits = lax.dot_general(
        x_ref[...], gw_ref[...], (((1,), (1,)), ((), ())),
        preferred_element_type=jnp.float32)
    logits = logits + gb_ref[...] + bias_ref[...]
    mx = jnp.max(logits, axis=1, keepdims=True)
    iota = lax.broadcasted_iota(jnp.int32, logits.shape, 1)
    eid_ref[...] = jnp.min(jnp.where(logits == mx, iota, E), axis=1,
                           keepdims=True)
    w_ref[...] = jax.nn.sigmoid(mx)


def _gating(xf, gate_w, gate_b, bias):
    t = xf.shape[0]
    return pl.pallas_call(
        _gate_body,
        out_shape=(
            jax.ShapeDtypeStruct((t, 1), jnp.int32),
            jax.ShapeDtypeStruct((t, 1), jnp.float32),
        ),
    )(xf, gate_w, gate_b.reshape(1, E), bias.reshape(1, E))


def _routing_meta(eid, t, g_max):
    onehot = (eid[:, None] == jnp.arange(E, dtype=jnp.int32)[None, :])
    counts = jnp.sum(onehot.astype(jnp.int32), axis=0)
    rank = jnp.take_along_axis(
        jnp.cumsum(onehot.astype(jnp.int32), axis=0), eid[:, None], 1)[:, 0] - 1
    nblk = (counts + BLK - 1) // BLK
    cs = jnp.cumsum(nblk)
    blk_start = jnp.concatenate([jnp.zeros((1,), jnp.int32), cs[:-1]])
    g_real = cs[-1]
    pos = blk_start[eid] * BLK + rank
    init = jnp.arange(g_max * BLK, dtype=jnp.int32) % t
    tok = init.at[pos].set(
        jnp.arange(t, dtype=jnp.int32), mode="drop", unique_indices=True)
    g_idx = jnp.arange(g_max, dtype=jnp.int32)
    be_raw = jnp.sum((g_idx[None, :] >= blk_start[:, None]).astype(jnp.int32),
                     axis=0) - 1
    last_e = jnp.max(jnp.where(counts > 0, jnp.arange(E, dtype=jnp.int32), -1))
    block_expert = jnp.where(g_idx < g_real, be_raw, last_e).astype(jnp.int32)
    prev_be = jnp.concatenate(
        [jnp.full((1,), -1, jnp.int32), block_expert[:-1]])
    is_start = ((g_idx < g_real) & (block_expert != prev_be)).astype(jnp.int32)
    run_id = jnp.cumsum(is_start) - 1
    nruns = run_id[-1] + 1
    run_expert = jnp.zeros((E + 1,), jnp.int32).at[
        jnp.where(is_start == 1, run_id, E)].set(block_expert, mode="drop")
    return (pos, tok, block_expert, g_real.astype(jnp.int32),
            run_id.astype(jnp.int32), is_start, run_expert,
            nruns.astype(jnp.int32))


def _sc_gather_rows(table, idx, m, rows_per_chunk):
    nw = 32
    per_w = m // nw
    chunks = per_w // rows_per_chunk
    mesh = plsc.VectorSubcoreMesh(core_axis_name="c", subcore_axis_name="s")

    @functools.partial(
        pl.kernel,
        mesh=mesh,
        out_type=jax.ShapeDtypeStruct((m, D), jnp.float32),
        scratch_types=[
            pltpu.VMEM((rows_per_chunk,), jnp.int32),
            pltpu.VMEM((rows_per_chunk, D), jnp.float32),
            pltpu.SemaphoreType.DMA,
        ],
    )
    def k(table_hbm, idx_hbm, out_hbm, idx_v, rows_v, sem):
        wid = lax.axis_index("s") * 2 + lax.axis_index("c")
        base = wid * per_w
        for c in range(chunks):
            off = base + c * rows_per_chunk
            pltpu.sync_copy(idx_hbm.at[pl.ds(off, rows_per_chunk)], idx_v)
            pltpu.async_copy(table_hbm.at[idx_v], rows_v, sem).wait()
            pltpu.sync_copy(rows_v, out_hbm.at[pl.ds(off, rows_per_chunk)])

    return k(table, idx)


N_SLOTS = 3


def _moe_body(be_ref, nblk_ref, rid_ref, st_ref, rex_ref, nr_ref,
              xp_ref, w1_hbm, b1_ref, w2_hbm, b2_ref, out_ref,
              w1_buf, w2_buf, sem1, sem2):
    g = pl.program_id(0)
    rid = rid_ref[g]
    slot = lax.rem(rid, N_SLOTS)
    e_cur = be_ref[g]

    def w_copy(e, s):
        return (pltpu.make_async_copy(w1_hbm.at[e], w1_buf.at[s],
                                      sem1.at[s]),
                pltpu.make_async_copy(w2_hbm.at[e], w2_buf.at[s],
                                      sem2.at[s]))

    @pl.when(g == 0)
    def _():
        c1, c2 = w_copy(e_cur, 0)
        c1.start()
        c2.start()

    @pl.when(st_ref[g] == 1)
    def _():
        c1, c2 = w_copy(e_cur, slot)
        c1.wait()
        c2.wait()
        nrid = rid + 1

        @pl.when(nrid < nr_ref[0])
        def _():
            n1, n2 = w_copy(rex_ref[nrid], lax.rem(nrid, N_SLOTS))
            n1.start()
            n2.start()

    @pl.when(g < nblk_ref[0])
    def _():
        xb = xp_ref[...]
        h = lax.dot_general(xb, w1_buf[slot], (((1,), (1,)), ((), ())),
                            preferred_element_type=jnp.float32)
        h = _gelu(h + b1_ref[0])
        y = lax.dot_general(h, w2_buf[slot], (((1,), (1,)), ((), ())),
                            preferred_element_type=jnp.float32)
        out_ref[...] = y + b2_ref[0]


def _grouped_mlp(x_padded, ew1, eb1, ew2, eb2, block_expert, g_real,
                 run_id, is_start, run_expert, nruns, g_max):
    grid_spec = pltpu.PrefetchScalarGridSpec(
        num_scalar_prefetch=6,
        grid=(g_max,),
        in_specs=[
            pl.BlockSpec((BLK, D), lambda g, *s: (g, 0)),
            pl.BlockSpec(memory_space=pl.ANY),
            pl.BlockSpec((1, 1, H), lambda g, be, nb, rid, st, rex, nr:
                         (be[g], 0, 0)),
            pl.BlockSpec(memory_space=pl.ANY),
            pl.BlockSpec((1, 1, D), lambda g, be, nb, rid, st, rex, nr:
                         (be[g], 0, 0)),
        ],
        out_specs=pl.BlockSpec((BLK, D), lambda g, *s: (g, 0)),
        scratch_shapes=[
            pltpu.VMEM((N_SLOTS, H, D), jnp.float32),
            pltpu.VMEM((N_SLOTS, D, H), jnp.float32),
            pltpu.SemaphoreType.DMA((N_SLOTS,)),
            pltpu.SemaphoreType.DMA((N_SLOTS,)),
        ],
    )
    return pl.pallas_call(
        _moe_body,
        grid_spec=grid_spec,
        out_shape=jax.ShapeDtypeStruct((g_max * BLK, D), jnp.float32),
    )(block_expert, g_real.reshape(1), run_id, is_start, run_expert,
      nruns.reshape(1), x_padded, ew1,
      eb1.reshape(E, 1, H), ew2, eb2.reshape(E, 1, D))


def _shared_body(x_ref, sw1_ref, sb1_ref, sw2_ref, sb2_ref, yt_ref, w_ref,
                 out_ref):
    h = lax.dot_general(x_ref[...], sw1_ref[...], (((1,), (1,)), ((), ())),
                        preferred_element_type=jnp.float32)
    h = _gelu(h + sb1_ref[...])
    y = lax.dot_general(h, sw2_ref[...], (((1,), (1,)), ((), ())),
                        preferred_element_type=jnp.float32)
    out_ref[...] = y + sb2_ref[...] + w_ref[...] * yt_ref[...]


def _shared_combine(xf, sw1, sb1, sw2, sb2, y_tok, w):
    t = xf.shape[0]
    tb = 256
    return pl.pallas_call(
        _shared_body,
        grid=(t // tb,),
        in_specs=[
            pl.BlockSpec((tb, D), lambda g: (g, 0)),
            pl.BlockSpec((H, D), lambda g: (0, 0)),
            pl.BlockSpec((1, H), lambda g: (0, 0)),
            pl.BlockSpec((D, H), lambda g: (0, 0)),
            pl.BlockSpec((1, D), lambda g: (0, 0)),
            pl.BlockSpec((tb, D), lambda g: (g, 0)),
            pl.BlockSpec((tb, 1), lambda g: (g, 0)),
        ],
        out_specs=pl.BlockSpec((tb, D), lambda g: (g, 0)),
        out_shape=jax.ShapeDtypeStruct((t, D), jnp.float32),
    )(xf, sw1, sb1.reshape(1, H), sw2, sb2.reshape(1, D), y_tok, w)


def kernel(x, gate_w, gate_b, bias, ew1, eb1, ew2, eb2, sw1, sb1, sw2, sb2):
    bs, var, ln, d = x.shape
    t = bs * var * ln
    g_max = E + t // BLK
    xf = x.reshape(t, d)

    eid2, w2d = _gating(xf, gate_w, gate_b, bias)
    eid = eid2[:, 0]
    (pos, tok, block_expert, g_real, run_id, is_start, run_expert,
     nruns) = _routing_meta(eid, t, g_max)

    x_padded = _sc_gather_rows(xf, tok, g_max * BLK, 64)
    y_padded = _grouped_mlp(x_padded, ew1, eb1, ew2, eb2, block_expert,
                            g_real, run_id, is_start, run_expert, nruns,
                            g_max)
    y_tok = _sc_gather_rows(y_padded, pos, t, 64)
    out = _shared_combine(xf, sw1, sb1, sw2, sb2, y_tok, w2d)
    return out.reshape(bs, var, ln, d)


# speedup vs baseline: 1.1449x; 1.1449x over previous
import functools

import jax
import jax.numpy as jnp
from jax import lax
from jax.experimental import pallas as pl
from jax.experimental.pallas import tpu as pltpu
from jax.experimental.pallas import tpu_sc as plsc

D = 768
H = 3072
E = 64
BLK = 64


def _gelu(v):
    return 0.5 * v * (1.0 + lax.erf(v * 0.7071067811865476))


def _gate_body(x_ref, gw_ref, gb_ref, bias_ref, eid_ref, w_ref):
    logits = lax.dot_general(
        x_ref[...], gw_ref[...], (((1,), (1,)), ((), ())),
        preferred_element_type=jnp.float32)
    logits = logits + gb_ref[...] + bias_ref[...]
    mx = jnp.max(logits, axis=1, keepdims=True)
    iota = lax.broadcasted_iota(jnp.int32, logits.shape, 1)
    eid_ref[...] = jnp.min(jnp.where(logits == mx, iota, E), axis=1,
                           keepdims=True)
    w_ref[...] = jax.nn.sigmoid(mx)


def _gating(xf, gate_w, gate_b, bias):
    t = xf.shape[0]
    return pl.pallas_call(
        _gate_body,
        out_shape=(
            jax.ShapeDtypeStruct((t, 1), jnp.int32),
            jax.ShapeDtypeStruct((t, 1), jnp.float32),
        ),
    )(xf, gate_w, gate_b.reshape(1, E), bias.reshape(1, E))


def _routing_meta(eid, t, g_max):
    onehot = (eid[:, None] == jnp.arange(E, dtype=jnp.int32)[None, :])
    counts = jnp.sum(onehot.astype(jnp.int32), axis=0)
    rank = jnp.take_along_axis(
        jnp.cumsum(onehot.astype(jnp.int32), axis=0), eid[:, None], 1)[:, 0] - 1
    nblk = (counts + BLK - 1) // BLK
    cs = jnp.cumsum(nblk)
    blk_start = jnp.concatenate([jnp.zeros((1,), jnp.int32), cs[:-1]])
    g_real = cs[-1]
    pos = blk_start[eid] * BLK + rank
    init = jnp.arange(g_max * BLK, dtype=jnp.int32) % t
    tok = init.at[pos].set(
        jnp.arange(t, dtype=jnp.int32), mode="drop", unique_indices=True)
    g_idx = jnp.arange(g_max, dtype=jnp.int32)
    be_raw = jnp.sum((g_idx[None, :] >= blk_start[:, None]).astype(jnp.int32),
                     axis=0) - 1
    last_e = jnp.max(jnp.where(counts > 0, jnp.arange(E, dtype=jnp.int32), -1))
    block_expert = jnp.where(g_idx < g_real, be_raw, last_e).astype(jnp.int32)
    return pos, tok, block_expert, g_real.astype(jnp.int32)


def _sc_gather_rows(table, idx, m, rows_per_chunk):
    nw = 32
    per_w = m // nw
    chunks = per_w // rows_per_chunk
    mesh = plsc.VectorSubcoreMesh(core_axis_name="c", subcore_axis_name="s")

    @functools.partial(
        pl.kernel,
        mesh=mesh,
        out_type=jax.ShapeDtypeStruct((m, D), jnp.float32),
        scratch_types=[
            pltpu.VMEM((rows_per_chunk,), jnp.int32),
            pltpu.VMEM((rows_per_chunk, D), jnp.float32),
            pltpu.SemaphoreType.DMA,
        ],
    )
    def k(table_hbm, idx_hbm, out_hbm, idx_v, rows_v, sem):
        wid = lax.axis_index("s") * 2 + lax.axis_index("c")
        base = wid * per_w
        for c in range(chunks):
            off = base + c * rows_per_chunk
            pltpu.sync_copy(idx_hbm.at[pl.ds(off, rows_per_chunk)], idx_v)
            pltpu.async_copy(table_hbm.at[idx_v], rows_v, sem).wait()
            pltpu.sync_copy(rows_v, out_hbm.at[pl.ds(off, rows_per_chunk)])

    return k(table, idx)


def _moe_body(be_ref, nblk_ref, xp_ref, w1_ref, b1_ref, w2_ref, b2_ref,
              out_ref):
    del be_ref
    g = pl.program_id(0)

    @pl.when(g < nblk_ref[0])
    def _():
        xb = xp_ref[...]
        h = lax.dot_general(xb, w1_ref[0], (((1,), (1,)), ((), ())),
                            preferred_element_type=jnp.float32)
        h = _gelu(h + b1_ref[0])
        y = lax.dot_general(h, w2_ref[0], (((1,), (1,)), ((), ())),
                            preferred_element_type=jnp.float32)
        out_ref[...] = y + b2_ref[0]


def _grouped_mlp(x_padded, ew1, eb1, ew2, eb2, block_expert, g_real, g_max):
    grid_spec = pltpu.PrefetchScalarGridSpec(
        num_scalar_prefetch=2,
        grid=(g_real,),
        in_specs=[
            pl.BlockSpec((BLK, D), lambda g, be, nb: (g, 0)),
            pl.BlockSpec((1, H, D), lambda g, be, nb: (be[g], 0, 0)),
            pl.BlockSpec((1, 1, H), lambda g, be, nb: (be[g], 0, 0)),
            pl.BlockSpec((1, D, H), lambda g, be, nb: (be[g], 0, 0)),
            pl.BlockSpec((1, 1, D), lambda g, be, nb: (be[g], 0, 0)),
        ],
        out_specs=pl.BlockSpec((BLK, D), lambda g, be, nb: (g, 0)),
    )
    return pl.pallas_call(
        _moe_body,
        grid_spec=grid_spec,
        out_shape=jax.ShapeDtypeStruct((g_max * BLK, D), jnp.float32),
    )(block_expert, g_real.reshape(1), x_padded, ew1,
      eb1.reshape(E, 1, H), ew2, eb2.reshape(E, 1, D))


def _shared_body(x_ref, sw1_ref, sb1_ref, sw2_ref, sb2_ref, yt_ref, w_ref,
                 out_ref):
    h = lax.dot_general(x_ref[...], sw1_ref[...], (((1,), (1,)), ((), ())),
                        preferred_element_type=jnp.float32)
    h = _gelu(h + sb1_ref[...])
    y = lax.dot_general(h, sw2_ref[...], (((1,), (1,)), ((), ())),
                        preferred_element_type=jnp.float32)
    out_ref[...] = y + sb2_ref[...] + w_ref[...] * yt_ref[...]


def _shared_combine(xf, sw1, sb1, sw2, sb2, y_tok, w):
    t = xf.shape[0]
    tb = 256
    return pl.pallas_call(
        _shared_body,
        grid=(t // tb,),
        in_specs=[
            pl.BlockSpec((tb, D), lambda g: (g, 0)),
            pl.BlockSpec((H, D), lambda g: (0, 0)),
            pl.BlockSpec((1, H), lambda g: (0, 0)),
            pl.BlockSpec((D, H), lambda g: (0, 0)),
            pl.BlockSpec((1, D), lambda g: (0, 0)),
            pl.BlockSpec((tb, D), lambda g: (g, 0)),
            pl.BlockSpec((tb, 1), lambda g: (g, 0)),
        ],
        out_specs=pl.BlockSpec((tb, D), lambda g: (g, 0)),
        out_shape=jax.ShapeDtypeStruct((t, D), jnp.float32),
    )(xf, sw1, sb1.reshape(1, H), sw2, sb2.reshape(1, D), y_tok, w)


def kernel(x, gate_w, gate_b, bias, ew1, eb1, ew2, eb2, sw1, sb1, sw2, sb2):
    bs, var, ln, d = x.shape
    t = bs * var * ln
    g_max = E + t // BLK
    xf = x.reshape(t, d)

    eid2, w2d = _gating(xf, gate_w, gate_b, bias)
    eid = eid2[:, 0]
    pos, tok, block_expert, g_real = _routing_meta(eid, t, g_max)

    x_padded = _sc_gather_rows(xf, tok, g_max * BLK, 64)
    y_padded = _grouped_mlp(x_padded, ew1, eb1, ew2, eb2, block_expert,
                            g_real, g_max)
    y_tok = _sc_gather_rows(y_padded, pos, t, 64)
    out = _shared_combine(xf, sw1, sb1, sw2, sb2, y_tok, w2d)
    return out.reshape(bs, var, ln, d)
